# Initial kernel scaffold; baseline (speedup 1.0000x reference)
import functools

import jax
import jax.numpy as jnp
from jax import lax
from jax.experimental import pallas as pl
from jax.experimental.pallas import tpu as pltpu
from jax.experimental.pallas import tpu_sc as plsc

N = 10000
E = 320000
D = 128
F32 = jnp.float32

NC = 2
NS = 16
NW = NC * NS
EPW = E // NW
CH = 80
NCHUNK = EPW // CH
RCH = 80
NRCH = N // RCH


def _dot(a, b):
    return jnp.dot(a, b, preferred_element_type=F32)


def _proj2_body(x_ref, w1_ref, w2_ref, o1_ref, o2_ref):
    x = x_ref[...]
    o1_ref[...] = _dot(x, w1_ref[...])
    o2_ref[...] = _dot(x, w2_ref[...])


def _proj2(x, w1, w2, blk):
    rows, k = x.shape
    grid = rows // blk
    return pl.pallas_call(
        _proj2_body,
        grid=(grid,),
        in_specs=[
            pl.BlockSpec((blk, k), lambda i: (i, 0)),
            pl.BlockSpec((k, D), lambda i: (0, 0)),
            pl.BlockSpec((k, D), lambda i: (0, 0)),
        ],
        out_specs=[
            pl.BlockSpec((blk, D), lambda i: (i, 0)),
            pl.BlockSpec((blk, D), lambda i: (i, 0)),
        ],
        out_shape=[
            jax.ShapeDtypeStruct((rows, D), F32),
            jax.ShapeDtypeStruct((rows, D), F32),
        ],
    )(x, w1, w2)


def _node_body(s0_ref, s1_ref, x_ref, we2_ref, wn1a_ref, wn1b_ref, wn2_ref,
               ws_ref, wd_ref, h_ref, a_ref, b_ref):
    red = _dot(s0_ref[...] + s1_ref[...], we2_ref[...])
    z = _dot(x_ref[...], wn1a_ref[...]) + _dot(red, wn1b_ref[...])
    h = _dot(jnp.maximum(z, 0.01 * z), wn2_ref[...])
    h_ref[...] = h
    a_ref[...] = _dot(h, ws_ref[...])
    b_ref[...] = _dot(h, wd_ref[...])


def _node(s0, s1, x, we2, wn1a, wn1b, wn2, ws, wd, blk=2000):
    grid = N // blk
    wspec = pl.BlockSpec((D, D), lambda i: (0, 0))
    rspec = pl.BlockSpec((blk, D), lambda i: (i, 0))
    return pl.pallas_call(
        _node_body,
        grid=(grid,),
        in_specs=[rspec, rspec, rspec] + [wspec] * 5,
        out_specs=[rspec, rspec, rspec],
        out_shape=[jax.ShapeDtypeStruct((N, D), F32)] * 3,
    )(s0, s1, x, we2, wn1a, wn1b, wn2, ws, wd)


def _final_body(s0_ref, s1_ref, x_ref, we2_ref, wn1a_ref, wn1b_ref, wn2_ref,
                o_ref):
    red = _dot(s0_ref[...] + s1_ref[...], we2_ref[...])
    z = _dot(x_ref[...], wn1a_ref[...]) + _dot(red, wn1b_ref[...])
    o_ref[...] = _dot(jnp.maximum(z, 0.01 * z), wn2_ref[...])


def _final(s0, s1, x, we2, wn1a, wn1b, wn2, blk=2000):
    grid = N // blk
    wspec = pl.BlockSpec((D, D), lambda i: (0, 0))
    rspec = pl.BlockSpec((blk, D), lambda i: (i, 0))
    return pl.pallas_call(
        _final_body,
        grid=(grid,),
        in_specs=[rspec, rspec, rspec] + [wspec] * 4,
        out_specs=rspec,
        out_shape=jax.ShapeDtypeStruct((N, D), F32),
    )(s0, s1, x, we2, wn1a, wn1b, wn2)


def _edge_pass(a_tbl, b_tbl, c_tbl, src, dst):
    mesh = plsc.VectorSubcoreMesh(core_axis_name="c", subcore_axis_name="s")

    @functools.partial(
        pl.kernel,
        mesh=mesh,
        out_type=(
            jax.ShapeDtypeStruct((N, D), F32),
            jax.ShapeDtypeStruct((N, D), F32),
        ),
        scratch_types=[
            pltpu.VMEM_SHARED((N, D), F32),
            pltpu.VMEM((CH, D), F32),
            pltpu.VMEM((CH, D), F32),
            pltpu.VMEM((CH, D), F32),
            pltpu.VMEM((CH,), jnp.int32),
            pltpu.VMEM((CH,), jnp.int32),
            pltpu.SemaphoreType.DMA,
            pltpu.SemaphoreType.DMA,
        ],
    )
    def k(a_hbm, b_hbm, c_hbm, src_hbm, dst_hbm, out0, out1,
          s_sh, buf_f, buf_r, bufc, src_v, dst_v, sem1, sem2):
        c = lax.axis_index("c")
        s = lax.axis_index("s")
        wid = s * NC + c

        def zbuf_body(g, _):
            r = g // (D // 16)
            j = (g % (D // 16)) * 16
            bufc[r, pl.ds(j, 16)] = jnp.zeros((16,), F32)
            return 0
        lax.fori_loop(0, RCH * (D // 16), zbuf_body, 0)

        z_lo = (NRCH * s) // NS
        z_hi = (NRCH * (s + 1)) // NS

        def zacc_body(t, _):
            pltpu.sync_copy(bufc, s_sh.at[pl.ds(t * RCH, RCH)])
            return 0
        lax.fori_loop(z_lo, z_hi, zacc_body, 0)
        plsc.subcore_barrier()

        base = wid * EPW

        def chunk_body(i, _):
            e0 = pl.multiple_of(base + i * CH, 8)
            pltpu.sync_copy(src_hbm.at[pl.ds(e0, CH)], src_v)
            pltpu.sync_copy(dst_hbm.at[pl.ds(e0, CH)], dst_v)
            pltpu.sync_copy(c_hbm.at[pl.ds(e0, CH)], bufc)
            pltpu.async_copy(a_hbm.at[src_v], buf_f, sem1).wait()
            pltpu.async_copy(b_hbm.at[dst_v], buf_f, sem1, add=True).wait()
            pltpu.async_copy(a_hbm.at[dst_v], buf_r, sem2).wait()
            pltpu.async_copy(b_hbm.at[src_v], buf_r, sem2, add=True).wait()

            def vec_body(g, _):
                r = g // (D // 16)
                j = (g % (D // 16)) * 16
                cc = bufc[r, pl.ds(j, 16)]
                zf = buf_f[r, pl.ds(j, 16)] + cc
                zr = buf_r[r, pl.ds(j, 16)] + cc
                buf_f[r, pl.ds(j, 16)] = jnp.maximum(zf, 0.01 * zf)
                buf_r[r, pl.ds(j, 16)] = jnp.maximum(zr, 0.01 * zr)
                return 0
            lax.fori_loop(0, CH * (D // 16), vec_body, 0)

            pltpu.sync_copy(buf_f, s_sh.at[dst_v], add=True)
            pltpu.sync_copy(buf_r, s_sh.at[src_v], add=True)
            return 0
        lax.fori_loop(0, NCHUNK, chunk_body, 0)
        plsc.subcore_barrier()

        def rd_body(t, _):
            rows = pl.ds(t * RCH, RCH)

            @pl.when(c == 0)
            def _():
                pltpu.sync_copy(s_sh.at[rows], out0.at[rows])

            @pl.when(c == 1)
            def _():
                pltpu.sync_copy(s_sh.at[rows], out1.at[rows])
            return 0
        lax.fori_loop(z_lo, z_hi, rd_body, 0)

    return k(a_tbl, b_tbl, c_tbl, src, dst)


def kernel(nf, ef, edge_index, we1_0, we2_0, wn1_0, wn2_0,
           we1_1, we2_1, wn1_1, wn2_1):
    fin = nf.shape[1]
    emb = wn2_0.shape[1]
    src = edge_index[0].astype(jnp.int32)
    dst = edge_index[1].astype(jnp.int32)

    a0, b0 = _proj2(nf, we1_0[:fin], we1_0[fin:2 * fin], blk=2000)
    c0, c1 = _proj2(ef, we1_0[2 * fin:], we1_1[2 * emb:], blk=8000)

    s0a, s0b = _edge_pass(a0, b0, c0, src, dst)

    h, a1, b1 = _node(s0a, s0b, nf, we2_0, wn1_0[:fin], wn1_0[fin:], wn2_0,
                      we1_1[:emb], we1_1[emb:2 * emb])

    s1a, s1b = _edge_pass(a1, b1, c1, src, dst)

    return _final(s1a, s1b, h, we2_1, wn1_1[:emb], wn1_1[emb:], wn2_1)


# speedup vs baseline: 4.3455x; 4.3455x over previous
import functools

import jax
import jax.numpy as jnp
from jax import lax
from jax.experimental import pallas as pl
from jax.experimental.pallas import tpu as pltpu
from jax.experimental.pallas import tpu_sc as plsc

N = 10000
E = 320000
D = 128
F32 = jnp.float32

NC = 2
NS = 16
NW = NC * NS
EPW = E // NW
CH = 80
NCHUNK = EPW // CH
RCH = 80
NRCH = N // RCH


def _dot(a, b):
    return jnp.dot(a, b, preferred_element_type=F32)


def _proj2_body(x_ref, w1_ref, w2_ref, o1_ref, o2_ref):
    x = x_ref[...]
    o1_ref[...] = _dot(x, w1_ref[...])
    o2_ref[...] = _dot(x, w2_ref[...])


def _proj2(x, w1, w2, blk):
    rows, k = x.shape
    grid = rows // blk
    return pl.pallas_call(
        _proj2_body,
        grid=(grid,),
        in_specs=[
            pl.BlockSpec((blk, k), lambda i: (i, 0)),
            pl.BlockSpec((k, D), lambda i: (0, 0)),
            pl.BlockSpec((k, D), lambda i: (0, 0)),
        ],
        out_specs=[
            pl.BlockSpec((blk, D), lambda i: (i, 0)),
            pl.BlockSpec((blk, D), lambda i: (i, 0)),
        ],
        out_shape=[
            jax.ShapeDtypeStruct((rows, D), F32),
            jax.ShapeDtypeStruct((rows, D), F32),
        ],
    )(x, w1, w2)


def _node_body(s0_ref, s1_ref, x_ref, we2_ref, wn1a_ref, wn1b_ref, wn2_ref,
               ws_ref, wd_ref, h_ref, a_ref, b_ref):
    red = _dot(s0_ref[...] + s1_ref[...], we2_ref[...])
    z = _dot(x_ref[...], wn1a_ref[...]) + _dot(red, wn1b_ref[...])
    h = _dot(jnp.maximum(z, 0.01 * z), wn2_ref[...])
    h_ref[...] = h
    a_ref[...] = _dot(h, ws_ref[...])
    b_ref[...] = _dot(h, wd_ref[...])


def _node(s0, s1, x, we2, wn1a, wn1b, wn2, ws, wd, blk=2000):
    grid = N // blk
    wspec = pl.BlockSpec((D, D), lambda i: (0, 0))
    rspec = pl.BlockSpec((blk, D), lambda i: (i, 0))
    return pl.pallas_call(
        _node_body,
        grid=(grid,),
        in_specs=[rspec, rspec, rspec] + [wspec] * 6,
        out_specs=[rspec, rspec, rspec],
        out_shape=[jax.ShapeDtypeStruct((N, D), F32)] * 3,
    )(s0, s1, x, we2, wn1a, wn1b, wn2, ws, wd)


def _final_body(s0_ref, s1_ref, x_ref, we2_ref, wn1a_ref, wn1b_ref, wn2_ref,
                o_ref):
    red = _dot(s0_ref[...] + s1_ref[...], we2_ref[...])
    z = _dot(x_ref[...], wn1a_ref[...]) + _dot(red, wn1b_ref[...])
    o_ref[...] = _dot(jnp.maximum(z, 0.01 * z), wn2_ref[...])


def _final(s0, s1, x, we2, wn1a, wn1b, wn2, blk=2000):
    grid = N // blk
    wspec = pl.BlockSpec((D, D), lambda i: (0, 0))
    rspec = pl.BlockSpec((blk, D), lambda i: (i, 0))
    return pl.pallas_call(
        _final_body,
        grid=(grid,),
        in_specs=[rspec, rspec, rspec] + [wspec] * 4,
        out_specs=rspec,
        out_shape=jax.ShapeDtypeStruct((N, D), F32),
    )(s0, s1, x, we2, wn1a, wn1b, wn2)


def _edge_pass(a_tbl, b_tbl, c_tbl, src, dst):
    mesh = plsc.VectorSubcoreMesh(core_axis_name="c", subcore_axis_name="s")

    @functools.partial(
        pl.kernel,
        mesh=mesh,
        out_type=(
            jax.ShapeDtypeStruct((N, D), F32),
            jax.ShapeDtypeStruct((N, D), F32),
        ),
        scratch_types=[
            pltpu.VMEM_SHARED((N, D), F32),
            pltpu.VMEM((CH, D), F32),
            pltpu.VMEM((CH, D), F32),
            pltpu.VMEM((CH, D), F32),
            pltpu.VMEM((CH,), jnp.int32),
            pltpu.VMEM((CH,), jnp.int32),
            pltpu.SemaphoreType.DMA,
            pltpu.SemaphoreType.DMA,
        ],
    )
    def k(a_hbm, b_hbm, c_hbm, src_hbm, dst_hbm, out0, out1,
          s_sh, buf_f, buf_r, bufc, src_v, dst_v, sem1, sem2):
        c = lax.axis_index("c")
        s = lax.axis_index("s")
        wid = s * NC + c

        def zbuf_body(g, _):
            r = g // (D // 16)
            j = (g % (D // 16)) * 16
            bufc[r, pl.ds(j, 16)] = jnp.zeros((16,), F32)
            return 0
        lax.fori_loop(0, RCH * (D // 16), zbuf_body, 0)

        z_lo = (NRCH * s) // NS
        z_hi = (NRCH * (s + 1)) // NS

        def zacc_body(t, _):
            pltpu.sync_copy(bufc, s_sh.at[pl.ds(t * RCH, RCH)])
            return 0
        lax.fori_loop(z_lo, z_hi, zacc_body, 0)
        plsc.subcore_barrier()

        base = wid * EPW

        def chunk_body(i, _):
            e0 = pl.multiple_of(base + i * CH, 8)
            pltpu.sync_copy(src_hbm.at[pl.ds(e0, CH)], src_v)
            pltpu.sync_copy(dst_hbm.at[pl.ds(e0, CH)], dst_v)
            pltpu.sync_copy(c_hbm.at[pl.ds(e0, CH)], bufc)
            pltpu.async_copy(a_hbm.at[src_v], buf_f, sem1).wait()
            pltpu.async_copy(b_hbm.at[dst_v], buf_f, sem1, add=True).wait()
            pltpu.async_copy(a_hbm.at[dst_v], buf_r, sem2).wait()
            pltpu.async_copy(b_hbm.at[src_v], buf_r, sem2, add=True).wait()

            def vec_body(g, _):
                r = g // (D // 16)
                j = (g % (D // 16)) * 16
                cc = bufc[r, pl.ds(j, 16)]
                zf = buf_f[r, pl.ds(j, 16)] + cc
                zr = buf_r[r, pl.ds(j, 16)] + cc
                buf_f[r, pl.ds(j, 16)] = jnp.maximum(zf, 0.01 * zf)
                buf_r[r, pl.ds(j, 16)] = jnp.maximum(zr, 0.01 * zr)
                return 0
            lax.fori_loop(0, CH * (D // 16), vec_body, 0)

            pltpu.sync_copy(buf_f, s_sh.at[dst_v], add=True)
            pltpu.sync_copy(buf_r, s_sh.at[src_v], add=True)
            return 0
        lax.fori_loop(0, NCHUNK, chunk_body, 0)
        plsc.subcore_barrier()

        def rd_body(t, _):
            rows = pl.ds(t * RCH, RCH)

            @pl.when(c == 0)
            def _():
                pltpu.sync_copy(s_sh.at[rows], out0.at[rows])

            @pl.when(c == 1)
            def _():
                pltpu.sync_copy(s_sh.at[rows], out1.at[rows])
            return 0
        lax.fori_loop(z_lo, z_hi, rd_body, 0)

    return k(a_tbl, b_tbl, c_tbl, src, dst)


def kernel(nf, ef, edge_index, we1_0, we2_0, wn1_0, wn2_0,
           we1_1, we2_1, wn1_1, wn2_1):
    fin = nf.shape[1]
    emb = wn2_0.shape[1]
    src = edge_index[0].astype(jnp.int32)
    dst = edge_index[1].astype(jnp.int32)

    a0, b0 = _proj2(nf, we1_0[:fin], we1_0[fin:2 * fin], blk=2000)
    c0, c1 = _proj2(ef, we1_0[2 * fin:], we1_1[2 * emb:], blk=8000)

    s0a, s0b = _edge_pass(a0, b0, c0, src, dst)

    h, a1, b1 = _node(s0a, s0b, nf, we2_0, wn1_0[:fin], wn1_0[fin:], wn2_0,
                      we1_1[:emb], we1_1[emb:2 * emb])

    s1a, s1b = _edge_pass(a1, b1, c1, src, dst)

    return _final(s1a, s1b, h, we2_1, wn1_1[:emb], wn1_1[emb:], wn2_1)
